# Initial kernel scaffold; baseline (speedup 1.0000x reference)
import jax
import jax.numpy as jnp
from jax.experimental import pallas as pl


def kernel(img_feats, pc):
    raise NotImplementedError("write your pallas kernel here")


# speedup vs baseline: 1.7657x; 1.7657x over previous
import functools

import jax
import jax.numpy as jnp
from jax import lax
from jax.experimental import pallas as pl
from jax.experimental.pallas import tpu as pltpu
from jax.experimental.pallas import tpu_sc as plsc

_S, _B, _C, _H, _W, _N = 3, 16, 192, 56, 56, 4096
_BN = _B * _N
_NC, _NS = 2, 16
_NW = _NC * _NS
_PTS = _BN // _NW
_CH = 16
_NCH = _PTS // _CH
_NV = _C // 16
_ROW3 = _S * _C


def _sc_body(table, px_h, py_h, pz_h, out_h,
             px_v, py_v, pz_v,
             r0, r1, r2, r3, r4, r5, r6, r7, r8, r9, r10, r11,
             wts_v, out_v, sem):
    rows = (r0, r1, r2, r3, r4, r5, r6, r7, r8, r9, r10, r11)
    wid = lax.axis_index("s") * _NC + lax.axis_index("c")
    base = wid * _PTS
    b = base // _N
    rowbase = b * (_H * _W)

    pltpu.sync_copy(px_h.at[pl.ds(base, _PTS)], px_v)
    pltpu.sync_copy(py_h.at[pl.ds(base, _PTS)], py_v)
    pltpu.sync_copy(pz_h.at[pl.ds(base, _PTS)], pz_v)

    def chunk(ch, carry):
        off = ch * _CH
        X = px_v[pl.ds(off, _CH)]
        Y = py_v[pl.ds(off, _CH)]
        Z = pz_v[pl.ds(off, _CH)]
        az = jnp.abs(Z)
        wq = 420.0 * X / az + 111.5
        hq = 420.0 * Y / az + 111.5
        wq = jnp.clip(wq, 0.0, 223.0)
        hq = jnp.clip(hq, 0.0, 223.0)
        x = wq / (223.0 / (_W - 1.0))
        y = hq / (223.0 / (_H - 1.0))
        xi1 = x.astype(jnp.int32)
        yi1 = y.astype(jnp.int32)
        x1 = xi1.astype(jnp.float32)
        y1 = yi1.astype(jnp.float32)
        xi2 = xi1 + jnp.where(x > x1, 1, 0).astype(jnp.int32)
        yi2 = yi1 + jnp.where(y > y1, 1, 0).astype(jnp.int32)
        x2 = xi2.astype(jnp.float32)
        y2 = yi2.astype(jnp.float32)
        wts_v[pl.ds(0, 16)] = x2 - x
        wts_v[pl.ds(16, 16)] = x - x1
        wts_v[pl.ds(32, 16)] = y2 - y
        wts_v[pl.ds(48, 16)] = y - y1
        r_11 = rowbase + xi1 * _W + yi1
        r_12 = rowbase + xi1 * _W + yi2
        r_21 = rowbase + xi2 * _W + yi1
        r_22 = rowbase + xi2 * _W + yi2

        copies = []
        for s in range(_S):
            soff = s * (_B * _H * _W)
            for k, r in enumerate((r_11, r_12, r_21, r_22)):
                copies.append(
                    pltpu.async_copy(table.at[r + soff], rows[s * 4 + k], sem))
        for c in copies:
            c.wait()

        def point(p, pc):
            pidx = jnp.full((16,), 0, dtype=jnp.int32) + p
            gx1 = plsc.load_gather(wts_v, [pidx])
            gx2 = plsc.load_gather(wts_v, [pidx + 16])
            gy1 = plsc.load_gather(wts_v, [pidx + 32])
            gy2 = plsc.load_gather(wts_v, [pidx + 48])
            for s in range(_S):
                for v in range(_NV):
                    sl = pl.ds(v * 16, 16)
                    q11 = rows[s * 4 + 0][p, sl]
                    q12 = rows[s * 4 + 1][p, sl]
                    q21 = rows[s * 4 + 2][p, sl]
                    q22 = rows[s * 4 + 3][p, sl]
                    out_v[p, pl.ds(s * _C + v * 16, 16)] = (
                        gx1 * (gy1 * q11 + gy2 * q12)
                        + gx2 * (gy1 * q21 + gy2 * q22))
            return pc

        lax.fori_loop(0, _CH, point, 0)
        pltpu.sync_copy(out_v, out_h.at[pl.ds(base + off, _CH)])
        return carry

    lax.fori_loop(0, _NCH, chunk, 0)


_sc_call = functools.partial(
    pl.kernel,
    out_type=jax.ShapeDtypeStruct((_BN, _ROW3), jnp.float32),
    mesh=plsc.VectorSubcoreMesh(core_axis_name="c", subcore_axis_name="s"),
    compiler_params=pltpu.CompilerParams(use_tc_tiling_on_sc=False, needs_layout_passes=False),
    scratch_types=(
        [pltpu.VMEM((_PTS,), jnp.float32)] * 3
        + [pltpu.VMEM((_CH, _C), jnp.float32)] * 12
        + [pltpu.VMEM((64,), jnp.float32),
           pltpu.VMEM((_CH, _ROW3), jnp.float32),
           pltpu.SemaphoreType.DMA]
    ),
)(_sc_body)


def kernel(img_feats, pc):
    s, b, c, h, w = img_feats.shape
    table = jnp.transpose(img_feats, (0, 1, 3, 4, 2)).reshape(s * b * h * w, c)
    px = pc[:, :, 0].reshape(-1)
    py = pc[:, :, 1].reshape(-1)
    pz = pc[:, :, 2].reshape(-1)
    out = _sc_call(table, px, py, pz)
    return out.reshape(b, _N, s * c)


# speedup vs baseline: 2.4129x; 1.3665x over previous
import functools

import jax
import jax.numpy as jnp
from jax import lax
from jax.experimental import pallas as pl
from jax.experimental.pallas import tpu as pltpu
from jax.experimental.pallas import tpu_sc as plsc

_S, _B, _C, _H, _W, _N = 3, 16, 192, 56, 56, 4096
_BN = _B * _N
_NC, _NS = 2, 16
_NW = _NC * _NS
_PTS = _BN // _NW
_CH = 16
_NCH = _PTS // _CH
_NV = _C // 16
_ROW3 = _S * _C


def _sc_body(table, px_h, py_h, pz_h, out_h,
             px_v, py_v, pz_v,
             rA0, rA1, rA2, rA3, rA4, rA5, rA6, rA7, rA8, rA9, rA10, rA11,
             rB0, rB1, rB2, rB3, rB4, rB5, rB6, rB7, rB8, rB9, rB10, rB11,
             wtsA, wtsB, out_v, semA, semB):
    rowsA = (rA0, rA1, rA2, rA3, rA4, rA5, rA6, rA7, rA8, rA9, rA10, rA11)
    rowsB = (rB0, rB1, rB2, rB3, rB4, rB5, rB6, rB7, rB8, rB9, rB10, rB11)
    wid = lax.axis_index("s") * _NC + lax.axis_index("c")
    base = wid * _PTS
    b = base // _N
    rowbase = b * (_H * _W)

    pltpu.sync_copy(px_h.at[pl.ds(base, _PTS)], px_v)
    pltpu.sync_copy(py_h.at[pl.ds(base, _PTS)], py_v)
    pltpu.sync_copy(pz_h.at[pl.ds(base, _PTS)], pz_v)

    def issue(ch, rows, wts, sem):
        off = ch * _CH
        X = px_v[pl.ds(off, _CH)]
        Y = py_v[pl.ds(off, _CH)]
        Z = pz_v[pl.ds(off, _CH)]
        az = jnp.abs(Z)
        wq = 420.0 * X / az + 111.5
        hq = 420.0 * Y / az + 111.5
        wq = jnp.clip(wq, 0.0, 223.0)
        hq = jnp.clip(hq, 0.0, 223.0)
        x = wq / (223.0 / (_W - 1.0))
        y = hq / (223.0 / (_H - 1.0))
        xi1 = x.astype(jnp.int32)
        yi1 = y.astype(jnp.int32)
        x1 = xi1.astype(jnp.float32)
        y1 = yi1.astype(jnp.float32)
        xi2 = xi1 + jnp.where(x > x1, 1, 0).astype(jnp.int32)
        yi2 = yi1 + jnp.where(y > y1, 1, 0).astype(jnp.int32)
        x2 = xi2.astype(jnp.float32)
        y2 = yi2.astype(jnp.float32)
        wts[pl.ds(0, 16)] = x2 - x
        wts[pl.ds(16, 16)] = x - x1
        wts[pl.ds(32, 16)] = y2 - y
        wts[pl.ds(48, 16)] = y - y1
        r_11 = rowbase + xi1 * _W + yi1
        r_12 = rowbase + xi1 * _W + yi2
        r_21 = rowbase + xi2 * _W + yi1
        r_22 = rowbase + xi2 * _W + yi2
        for s in range(_S):
            soff = s * (_B * _H * _W)
            for k, r in enumerate((r_11, r_12, r_21, r_22)):
                pltpu.async_copy(table.at[r + soff], rows[s * 4 + k], sem)

    def drain(rows, sem):
        for k in range(12):
            pltpu.make_async_copy(table.at[pl.ds(0, _CH)], rows[k], sem).wait()

    def compute(ch, rows, wts):
        def point(p, pc):
            pidx = jnp.full((16,), 0, dtype=jnp.int32) + p
            gx1 = plsc.load_gather(wts, [pidx])
            gx2 = plsc.load_gather(wts, [pidx + 16])
            gy1 = plsc.load_gather(wts, [pidx + 32])
            gy2 = plsc.load_gather(wts, [pidx + 48])
            for s in range(_S):
                for v in range(_NV):
                    sl = pl.ds(v * 16, 16)
                    q11 = rows[s * 4 + 0][p, sl]
                    q12 = rows[s * 4 + 1][p, sl]
                    q21 = rows[s * 4 + 2][p, sl]
                    q22 = rows[s * 4 + 3][p, sl]
                    out_v[p, pl.ds(s * _C + v * 16, 16)] = (
                        gx1 * (gy1 * q11 + gy2 * q12)
                        + gx2 * (gy1 * q21 + gy2 * q22))
            return pc

        lax.fori_loop(0, _CH, point, 0)
        pltpu.sync_copy(out_v, out_h.at[pl.ds(base + ch * _CH, _CH)])

    issue(0, rowsA, wtsA, semA)

    def pair(i, carry):
        issue(2 * i + 1, rowsB, wtsB, semB)
        drain(rowsA, semA)
        compute(2 * i, rowsA, wtsA)

        @pl.when(i < _NCH // 2 - 1)
        def _():
            issue(2 * i + 2, rowsA, wtsA, semA)

        drain(rowsB, semB)
        compute(2 * i + 1, rowsB, wtsB)
        return carry

    lax.fori_loop(0, _NCH // 2, pair, 0)


_sc_call = functools.partial(
    pl.kernel,
    out_type=jax.ShapeDtypeStruct((_BN, _ROW3), jnp.float32),
    mesh=plsc.VectorSubcoreMesh(core_axis_name="c", subcore_axis_name="s"),
    compiler_params=pltpu.CompilerParams(
        use_tc_tiling_on_sc=False, needs_layout_passes=False),
    scratch_types=(
        [pltpu.VMEM((_PTS,), jnp.float32)] * 3
        + [pltpu.VMEM((_CH, _C), jnp.float32)] * 24
        + [pltpu.VMEM((64,), jnp.float32)] * 2
        + [pltpu.VMEM((_CH, _ROW3), jnp.float32),
           pltpu.SemaphoreType.DMA, pltpu.SemaphoreType.DMA]
    ),
)(_sc_body)


def kernel(img_feats, pc):
    s, b, c, h, w = img_feats.shape
    table = jnp.transpose(img_feats, (0, 1, 3, 4, 2)).reshape(s * b * h * w, c)
    px = pc[:, :, 0].reshape(-1)
    py = pc[:, :, 1].reshape(-1)
    pz = pc[:, :, 2].reshape(-1)
    out = _sc_call(table, px, py, pz)
    return out.reshape(b, _N, s * c)


# speedup vs baseline: 2.4439x; 1.0129x over previous
import functools

import jax
import jax.numpy as jnp
from jax import lax
from jax.experimental import pallas as pl
from jax.experimental.pallas import tpu as pltpu
from jax.experimental.pallas import tpu_sc as plsc

_S, _B, _C, _H, _W, _N = 3, 16, 192, 56, 56, 4096
_BN = _B * _N
_NC, _NS = 2, 16
_NW = _NC * _NS
_PTS = _BN // _NW
_CH = 16
_NCH = _PTS // _CH
_NV = _C // 16
_ROW3 = _S * _C


def _sc_body(table, px_h, py_h, pz_h, out_h,
             px_v, py_v, pz_v,
             idxA1, idxA2, idxB1, idxB2,
             rowsA1, rowsA2, rowsB1, rowsB2,
             wtsA, wtsB, outA, outB,
             semA, semB, osem):
    wid = lax.axis_index("s") * _NC + lax.axis_index("c")
    base = wid * _PTS
    b = base // _N
    rowbase = b * (_H * _W)

    pltpu.sync_copy(px_h.at[pl.ds(base, _PTS)], px_v)
    pltpu.sync_copy(py_h.at[pl.ds(base, _PTS)], py_v)
    pltpu.sync_copy(pz_h.at[pl.ds(base, _PTS)], pz_v)

    def issue(ch, idx1, idx2, rows1, rows2, wts, sem):
        off = ch * _CH
        X = px_v[pl.ds(off, _CH)]
        Y = py_v[pl.ds(off, _CH)]
        Z = pz_v[pl.ds(off, _CH)]
        az = jnp.abs(Z)
        wq = 420.0 * X / az + 111.5
        hq = 420.0 * Y / az + 111.5
        wq = jnp.clip(wq, 0.0, 223.0)
        hq = jnp.clip(hq, 0.0, 223.0)
        x = wq / (223.0 / (_W - 1.0))
        y = hq / (223.0 / (_H - 1.0))
        xi1 = x.astype(jnp.int32)
        yi1 = y.astype(jnp.int32)
        x1 = xi1.astype(jnp.float32)
        y1 = yi1.astype(jnp.float32)
        xi2 = xi1 + jnp.where(x > x1, 1, 0).astype(jnp.int32)
        yi2 = yi1 + jnp.where(y > y1, 1, 0).astype(jnp.int32)
        x2 = xi2.astype(jnp.float32)
        y2 = yi2.astype(jnp.float32)
        wts[pl.ds(0, 16)] = x2 - x
        wts[pl.ds(16, 16)] = x - x1
        wts[pl.ds(32, 16)] = y2 - y
        wts[pl.ds(48, 16)] = y - y1
        r_11 = rowbase + xi1 * _W + yi1
        r_12 = rowbase + xi1 * _W + yi2
        r_21 = rowbase + xi2 * _W + yi1
        r_22 = rowbase + xi2 * _W + yi2
        for s in range(_S):
            soff = s * (_B * _H * _W)
            for k, r in enumerate((r_11, r_12, r_21, r_22)):
                j = s * 4 + k
                if j < 8:
                    idx1[pl.ds(j * 16, 16)] = r + soff
                else:
                    idx2[pl.ds((j - 8) * 16, 16)] = r + soff
        pltpu.async_copy(table.at[idx1], rows1, sem)
        pltpu.async_copy(table.at[idx2], rows2, sem)

    def drain(rows1, rows2, sem):
        pltpu.make_async_copy(table.at[pl.ds(0, 8 * _CH)], rows1, sem).wait()
        pltpu.make_async_copy(table.at[pl.ds(0, 4 * _CH)], rows2, sem).wait()

    def compute(ch, rows1, rows2, wts, out_v):
        def point(p, pc):
            pidx = jnp.full((16,), 0, dtype=jnp.int32) + p
            gx1 = plsc.load_gather(wts, [pidx])
            gx2 = plsc.load_gather(wts, [pidx + 16])
            gy1 = plsc.load_gather(wts, [pidx + 32])
            gy2 = plsc.load_gather(wts, [pidx + 48])
            for s in range(_S):
                for v in range(_NV):
                    sl = pl.ds(v * 16, 16)
                    if s < 2:
                        q11 = rows1[(s * 4 + 0) * 16 + p, sl]
                        q12 = rows1[(s * 4 + 1) * 16 + p, sl]
                        q21 = rows1[(s * 4 + 2) * 16 + p, sl]
                        q22 = rows1[(s * 4 + 3) * 16 + p, sl]
                    else:
                        q11 = rows2[0 * 16 + p, sl]
                        q12 = rows2[1 * 16 + p, sl]
                        q21 = rows2[2 * 16 + p, sl]
                        q22 = rows2[3 * 16 + p, sl]
                    out_v[p, pl.ds(s * _C + v * 16, 16)] = (
                        gx1 * (gy1 * q11 + gy2 * q12)
                        + gx2 * (gy1 * q21 + gy2 * q22))
            return pc

        lax.fori_loop(0, _CH, point, 0)
        pltpu.async_copy(out_v, out_h.at[pl.ds(base + ch * _CH, _CH)], osem)

    issue(0, idxA1, idxA2, rowsA1, rowsA2, wtsA, semA)

    def pair(i, carry):
        @pl.when(i > 0)
        def _():
            pltpu.make_async_copy(table.at[pl.ds(0, _CH)], outA, osem).wait()
            pltpu.make_async_copy(table.at[pl.ds(0, _CH)], outB, osem).wait()

        issue(2 * i + 1, idxB1, idxB2, rowsB1, rowsB2, wtsB, semB)
        drain(rowsA1, rowsA2, semA)
        compute(2 * i, rowsA1, rowsA2, wtsA, outA)

        @pl.when(i < _NCH // 2 - 1)
        def _():
            issue(2 * i + 2, idxA1, idxA2, rowsA1, rowsA2, wtsA, semA)

        drain(rowsB1, rowsB2, semB)
        compute(2 * i + 1, rowsB1, rowsB2, wtsB, outB)
        return carry

    lax.fori_loop(0, _NCH // 2, pair, 0)
    pltpu.make_async_copy(table.at[pl.ds(0, _CH)], outA, osem).wait()
    pltpu.make_async_copy(table.at[pl.ds(0, _CH)], outB, osem).wait()


_sc_call = functools.partial(
    pl.kernel,
    out_type=jax.ShapeDtypeStruct((_BN, _ROW3), jnp.float32),
    mesh=plsc.VectorSubcoreMesh(core_axis_name="c", subcore_axis_name="s"),
    compiler_params=pltpu.CompilerParams(
        use_tc_tiling_on_sc=False, needs_layout_passes=False),
    scratch_types=(
        [pltpu.VMEM((_PTS,), jnp.float32)] * 3
        + [pltpu.VMEM((8 * _CH,), jnp.int32),
           pltpu.VMEM((4 * _CH,), jnp.int32)] * 2
        + [pltpu.VMEM((8 * _CH, _C), jnp.float32),
           pltpu.VMEM((4 * _CH, _C), jnp.float32)] * 2
        + [pltpu.VMEM((64,), jnp.float32)] * 2
        + [pltpu.VMEM((_CH, _ROW3), jnp.float32)] * 2
        + [pltpu.SemaphoreType.DMA] * 3
    ),
)(_sc_body)


def kernel(img_feats, pc):
    s, b, c, h, w = img_feats.shape
    table = jnp.transpose(img_feats, (0, 1, 3, 4, 2)).reshape(s * b * h * w, c)
    px = pc[:, :, 0].reshape(-1)
    py = pc[:, :, 1].reshape(-1)
    pz = pc[:, :, 2].reshape(-1)
    out = _sc_call(table, px, py, pz)
    return out.reshape(b, _N, s * c)


# speedup vs baseline: 2.5640x; 1.0491x over previous
import functools

import jax
import jax.numpy as jnp
from jax import lax
from jax.experimental import pallas as pl
from jax.experimental.pallas import tpu as pltpu
from jax.experimental.pallas import tpu_sc as plsc

_S, _B, _C, _H, _W, _N = 3, 16, 192, 56, 56, 4096
_BN = _B * _N
_NC, _NS = 2, 16
_NW = _NC * _NS
_PTS = _BN // _NW
_CH = 16
_NCH = _PTS // _CH
_NV = _C // 16
_ROW3 = _S * _C


def _sc_body(table, px_h, py_h, pz_h, out_h,
             px_v, py_v, pz_v,
             idxA1, idxA2, idxB1, idxB2,
             rowsA1, rowsA2, rowsB1, rowsB2,
             wtsA, wtsB, outA, outB,
             semA, semB, osem):
    wid = lax.axis_index("s") * _NC + lax.axis_index("c")
    base = wid * _PTS
    b = base // _N
    rowbase = b * (_H * _W)

    pltpu.sync_copy(px_h.at[pl.ds(base, _PTS)], px_v)
    pltpu.sync_copy(py_h.at[pl.ds(base, _PTS)], py_v)
    pltpu.sync_copy(pz_h.at[pl.ds(base, _PTS)], pz_v)

    def issue(ch, idx1, idx2, rows1, rows2, wts, sem):
        off = ch * _CH
        X = px_v[pl.ds(off, _CH)]
        Y = py_v[pl.ds(off, _CH)]
        Z = pz_v[pl.ds(off, _CH)]
        az = jnp.abs(Z)
        wq = 420.0 * X / az + 111.5
        hq = 420.0 * Y / az + 111.5
        wq = jnp.clip(wq, 0.0, 223.0)
        hq = jnp.clip(hq, 0.0, 223.0)
        x = wq / (223.0 / (_W - 1.0))
        y = hq / (223.0 / (_H - 1.0))
        xi1 = x.astype(jnp.int32)
        yi1 = y.astype(jnp.int32)
        x1 = xi1.astype(jnp.float32)
        y1 = yi1.astype(jnp.float32)
        xi2 = xi1 + jnp.where(x > x1, 1, 0).astype(jnp.int32)
        yi2 = yi1 + jnp.where(y > y1, 1, 0).astype(jnp.int32)
        x2 = xi2.astype(jnp.float32)
        y2 = yi2.astype(jnp.float32)
        gx1 = x2 - x
        gx2 = x - x1
        gy1 = y2 - y
        gy2 = y - y1
        wts[pl.ds(0, 16)] = gx1 * gy1
        wts[pl.ds(16, 16)] = gx1 * gy2
        wts[pl.ds(32, 16)] = gx2 * gy1
        wts[pl.ds(48, 16)] = gx2 * gy2
        r_11 = rowbase + xi1 * _W + yi1
        r_12 = rowbase + xi1 * _W + yi2
        r_21 = rowbase + xi2 * _W + yi1
        r_22 = rowbase + xi2 * _W + yi2
        for s in range(_S):
            soff = s * (_B * _H * _W)
            for k, r in enumerate((r_11, r_12, r_21, r_22)):
                j = s * 4 + k
                if j < 8:
                    idx1[pl.ds(j * 16, 16)] = r + soff
                else:
                    idx2[pl.ds((j - 8) * 16, 16)] = r + soff
        pltpu.async_copy(table.at[idx1], rows1, sem)
        pltpu.async_copy(table.at[idx2], rows2, sem)

    def drain(rows1, rows2, sem):
        pltpu.make_async_copy(table.at[pl.ds(0, 8 * _CH)], rows1, sem).wait()
        pltpu.make_async_copy(table.at[pl.ds(0, 4 * _CH)], rows2, sem).wait()

    def compute(ch, rows1, rows2, wts, out_v):
        def point(p, pc):
            pidx = jnp.full((16,), 0, dtype=jnp.int32) + p
            w11 = plsc.load_gather(wts, [pidx])
            w12 = plsc.load_gather(wts, [pidx + 16])
            w21 = plsc.load_gather(wts, [pidx + 32])
            w22 = plsc.load_gather(wts, [pidx + 48])

            def corners(s, v):
                sl = pl.ds(v * 16, 16)
                if s < 2:
                    return (rows1[(s * 4 + 0) * 16 + p, sl],
                            rows1[(s * 4 + 1) * 16 + p, sl],
                            rows1[(s * 4 + 2) * 16 + p, sl],
                            rows1[(s * 4 + 3) * 16 + p, sl])
                return (rows2[0 * 16 + p, sl],
                        rows2[1 * 16 + p, sl],
                        rows2[2 * 16 + p, sl],
                        rows2[3 * 16 + p, sl])

            sv = [(s, v) for s in range(_S) for v in range(_NV)]
            for t in range(0, len(sv), 2):
                sa, va = sv[t]
                sb, vb = sv[t + 1]
                qa = corners(sa, va)
                qb = corners(sb, vb)
                ra = (qa[0] * w11 + qa[1] * w12) + (qa[2] * w21 + qa[3] * w22)
                rb = (qb[0] * w11 + qb[1] * w12) + (qb[2] * w21 + qb[3] * w22)
                out_v[p, pl.ds(sa * _C + va * 16, 16)] = ra
                out_v[p, pl.ds(sb * _C + vb * 16, 16)] = rb
            return pc

        lax.fori_loop(0, _CH, point, 0)
        pltpu.async_copy(out_v, out_h.at[pl.ds(base + ch * _CH, _CH)], osem)

    issue(0, idxA1, idxA2, rowsA1, rowsA2, wtsA, semA)

    def pair(i, carry):
        @pl.when(i > 0)
        def _():
            pltpu.make_async_copy(table.at[pl.ds(0, _CH)], outA, osem).wait()
            pltpu.make_async_copy(table.at[pl.ds(0, _CH)], outB, osem).wait()

        issue(2 * i + 1, idxB1, idxB2, rowsB1, rowsB2, wtsB, semB)
        drain(rowsA1, rowsA2, semA)
        compute(2 * i, rowsA1, rowsA2, wtsA, outA)

        @pl.when(i < _NCH // 2 - 1)
        def _():
            issue(2 * i + 2, idxA1, idxA2, rowsA1, rowsA2, wtsA, semA)

        drain(rowsB1, rowsB2, semB)
        compute(2 * i + 1, rowsB1, rowsB2, wtsB, outB)
        return carry

    lax.fori_loop(0, _NCH // 2, pair, 0)
    pltpu.make_async_copy(table.at[pl.ds(0, _CH)], outA, osem).wait()
    pltpu.make_async_copy(table.at[pl.ds(0, _CH)], outB, osem).wait()


_sc_call = functools.partial(
    pl.kernel,
    out_type=jax.ShapeDtypeStruct((_BN, _ROW3), jnp.float32),
    mesh=plsc.VectorSubcoreMesh(core_axis_name="c", subcore_axis_name="s"),
    compiler_params=pltpu.CompilerParams(
        use_tc_tiling_on_sc=False, needs_layout_passes=False),
    scratch_types=(
        [pltpu.VMEM((_PTS,), jnp.float32)] * 3
        + [pltpu.VMEM((8 * _CH,), jnp.int32),
           pltpu.VMEM((4 * _CH,), jnp.int32)] * 2
        + [pltpu.VMEM((8 * _CH, _C), jnp.float32),
           pltpu.VMEM((4 * _CH, _C), jnp.float32)] * 2
        + [pltpu.VMEM((64,), jnp.float32)] * 2
        + [pltpu.VMEM((_CH, _ROW3), jnp.float32)] * 2
        + [pltpu.SemaphoreType.DMA] * 3
    ),
)(_sc_body)


def kernel(img_feats, pc):
    s, b, c, h, w = img_feats.shape
    table = jnp.transpose(img_feats, (0, 1, 3, 4, 2)).reshape(s * b * h * w, c)
    px = pc[:, :, 0].reshape(-1)
    py = pc[:, :, 1].reshape(-1)
    pz = pc[:, :, 2].reshape(-1)
    out = _sc_call(table, px, py, pz)
    return out.reshape(b, _N, s * c)


# speedup vs baseline: 2.6226x; 1.0228x over previous
import functools

import jax
import jax.numpy as jnp
from jax import lax
from jax.experimental import pallas as pl
from jax.experimental.pallas import tpu as pltpu
from jax.experimental.pallas import tpu_sc as plsc

_S, _B, _C, _H, _W, _N = 3, 16, 192, 56, 56, 4096
_BN = _B * _N
_NC, _NS = 2, 16
_NW = _NC * _NS
_PTS = _BN // _NW
_CH = 16
_NCH = _PTS // _CH
_NV = _C // 16
_ROW3 = _S * _C


def _sc_body(table, px_h, py_h, pz_h, out_h,
             px_v, py_v, pz_v,
             idxA1, idxA2, idxB1, idxB2,
             rowsA1, rowsA2, rowsB1, rowsB2,
             wtsA, wtsB, outA, outB,
             semA, semB, osem):
    wid = lax.axis_index("s") * _NC + lax.axis_index("c")
    base = wid * _PTS
    b = base // _N
    rowbase = b * (_H * _W)

    pltpu.sync_copy(px_h.at[pl.ds(base, _PTS)], px_v)
    pltpu.sync_copy(py_h.at[pl.ds(base, _PTS)], py_v)
    pltpu.sync_copy(pz_h.at[pl.ds(base, _PTS)], pz_v)

    def issue(ch, idx1, idx2, rows1, rows2, wts, sem):
        off = ch * _CH
        X = px_v[pl.ds(off, _CH)]
        Y = py_v[pl.ds(off, _CH)]
        Z = pz_v[pl.ds(off, _CH)]
        az = jnp.abs(Z)
        wq = 420.0 * X / az + 111.5
        hq = 420.0 * Y / az + 111.5
        wq = jnp.clip(wq, 0.0, 223.0)
        hq = jnp.clip(hq, 0.0, 223.0)
        x = wq / (223.0 / (_W - 1.0))
        y = hq / (223.0 / (_H - 1.0))
        xi1 = x.astype(jnp.int32)
        yi1 = y.astype(jnp.int32)
        x1 = xi1.astype(jnp.float32)
        y1 = yi1.astype(jnp.float32)
        xi2 = xi1 + jnp.where(x > x1, 1, 0).astype(jnp.int32)
        yi2 = yi1 + jnp.where(y > y1, 1, 0).astype(jnp.int32)
        x2 = xi2.astype(jnp.float32)
        y2 = yi2.astype(jnp.float32)
        gx1 = x2 - x
        gx2 = x - x1
        gy1 = y2 - y
        gy2 = y - y1
        wts[pl.ds(0, 16)] = gx1 * gy1
        wts[pl.ds(16, 16)] = gx1 * gy2
        wts[pl.ds(32, 16)] = gx2 * gy1
        wts[pl.ds(48, 16)] = gx2 * gy2
        r_11 = rowbase + xi1 * _W + yi1
        r_12 = rowbase + xi1 * _W + yi2
        r_21 = rowbase + xi2 * _W + yi1
        r_22 = rowbase + xi2 * _W + yi2
        for s in range(_S):
            soff = s * (_B * _H * _W)
            for k, r in enumerate((r_11, r_12, r_21, r_22)):
                j = s * 4 + k
                if j < 8:
                    idx1[pl.ds(j * 16, 16)] = r + soff
                else:
                    idx2[pl.ds((j - 8) * 16, 16)] = r + soff
        pltpu.async_copy(table.at[idx1], rows1, sem)
        pltpu.async_copy(table.at[idx2], rows2, sem)

    def drain(rows1, rows2, sem):
        pltpu.make_async_copy(table.at[pl.ds(0, 8 * _CH)], rows1, sem).wait()
        pltpu.make_async_copy(table.at[pl.ds(0, 4 * _CH)], rows2, sem).wait()

    def compute(ch, rows1, rows2, wts, out_v):
        def point(p, pc):
            pidx = jnp.full((16,), 0, dtype=jnp.int32) + p
            w11 = plsc.load_gather(wts, [pidx])
            w12 = plsc.load_gather(wts, [pidx + 16])
            w21 = plsc.load_gather(wts, [pidx + 32])
            w22 = plsc.load_gather(wts, [pidx + 48])

            def corners(s, v):
                sl = pl.ds(v * 16, 16)
                if s < 2:
                    return (rows1[(s * 4 + 0) * 16 + p, sl],
                            rows1[(s * 4 + 1) * 16 + p, sl],
                            rows1[(s * 4 + 2) * 16 + p, sl],
                            rows1[(s * 4 + 3) * 16 + p, sl])
                return (rows2[0 * 16 + p, sl],
                        rows2[1 * 16 + p, sl],
                        rows2[2 * 16 + p, sl],
                        rows2[3 * 16 + p, sl])

            sv = [(s, v) for s in range(_S) for v in range(_NV)]

            def emit(pair_q, pair_sv):
                (qa, qb), ((sa, va), (sb, vb)) = pair_q, pair_sv
                ra = (qa[0] * w11 + qa[1] * w12) + (qa[2] * w21 + qa[3] * w22)
                rb = (qb[0] * w11 + qb[1] * w12) + (qb[2] * w21 + qb[3] * w22)
                out_v[p, pl.ds(sa * _C + va * 16, 16)] = ra
                out_v[p, pl.ds(sb * _C + vb * 16, 16)] = rb

            prev_q = (corners(*sv[0]), corners(*sv[1]))
            prev_sv = (sv[0], sv[1])
            for t in range(2, len(sv), 2):
                cur_q = (corners(*sv[t]), corners(*sv[t + 1]))
                emit(prev_q, prev_sv)
                prev_q, prev_sv = cur_q, (sv[t], sv[t + 1])
            emit(prev_q, prev_sv)
            return pc

        lax.fori_loop(0, _CH, point, 0)
        pltpu.async_copy(out_v, out_h.at[pl.ds(base + ch * _CH, _CH)], osem)

    issue(0, idxA1, idxA2, rowsA1, rowsA2, wtsA, semA)

    def pair(i, carry):
        @pl.when(i > 0)
        def _():
            pltpu.make_async_copy(table.at[pl.ds(0, _CH)], outA, osem).wait()
            pltpu.make_async_copy(table.at[pl.ds(0, _CH)], outB, osem).wait()

        issue(2 * i + 1, idxB1, idxB2, rowsB1, rowsB2, wtsB, semB)
        drain(rowsA1, rowsA2, semA)
        compute(2 * i, rowsA1, rowsA2, wtsA, outA)

        @pl.when(i < _NCH // 2 - 1)
        def _():
            issue(2 * i + 2, idxA1, idxA2, rowsA1, rowsA2, wtsA, semA)

        drain(rowsB1, rowsB2, semB)
        compute(2 * i + 1, rowsB1, rowsB2, wtsB, outB)
        return carry

    lax.fori_loop(0, _NCH // 2, pair, 0)
    pltpu.make_async_copy(table.at[pl.ds(0, _CH)], outA, osem).wait()
    pltpu.make_async_copy(table.at[pl.ds(0, _CH)], outB, osem).wait()


_sc_call = functools.partial(
    pl.kernel,
    out_type=jax.ShapeDtypeStruct((_BN, _ROW3), jnp.float32),
    mesh=plsc.VectorSubcoreMesh(core_axis_name="c", subcore_axis_name="s"),
    compiler_params=pltpu.CompilerParams(
        use_tc_tiling_on_sc=False, needs_layout_passes=False),
    scratch_types=(
        [pltpu.VMEM((_PTS,), jnp.float32)] * 3
        + [pltpu.VMEM((8 * _CH,), jnp.int32),
           pltpu.VMEM((4 * _CH,), jnp.int32)] * 2
        + [pltpu.VMEM((8 * _CH, _C), jnp.float32),
           pltpu.VMEM((4 * _CH, _C), jnp.float32)] * 2
        + [pltpu.VMEM((64,), jnp.float32)] * 2
        + [pltpu.VMEM((_CH, _ROW3), jnp.float32)] * 2
        + [pltpu.SemaphoreType.DMA] * 3
    ),
)(_sc_body)


def kernel(img_feats, pc):
    s, b, c, h, w = img_feats.shape
    table = jnp.transpose(img_feats, (0, 1, 3, 4, 2)).reshape(s * b * h * w, c)
    px = pc[:, :, 0].reshape(-1)
    py = pc[:, :, 1].reshape(-1)
    pz = pc[:, :, 2].reshape(-1)
    out = _sc_call(table, px, py, pz)
    return out.reshape(b, _N, s * c)
